# baseline (speedup 1.0000x reference)
import functools

import jax
import jax.numpy as jnp
from jax import lax
from jax.experimental import pallas as pl
from jax.experimental.pallas import tpu as pltpu
from jax.experimental.pallas import tpu_sc as plsc

_SEQ_LENS = (2048, 1792, 1536, 1280, 1024, 768, 512, 256)
_B = 8
_MAX = 2048
_D = 1024
_NC = 2
_NS = 16
_NW = _NC * _NS
_WROWS = _B * _MAX // _NW
_ROWS = 120
_CR = 40


def _pad_body(x0, x1, x2, x3, x4, x5, x6, x7, out, big, sem_z, sem_in,
              sem_out):
    xs = (x0, x1, x2, x3, x4, x5, x6, x7)
    wid = lax.axis_index("s") * _NC + lax.axis_index("c")
    bb = wid // 4
    q = wid % 4
    jlo = q * _WROWS

    def _fill_zero(row0, nrows):
        def _row(r, carry):
            for cc in range(_D // 16):
                big[r, pl.ds(cc * 16, 16)] = jnp.zeros((16,), jnp.float32)
            return carry

        lax.fori_loop(row0, row0 + nrows, _row, 0)

    def _drain(sem, nrows):
        pltpu.make_async_copy(
            x0.at[pl.ds(0, nrows), :], big.at[pl.ds(0, nrows), :], sem
        ).wait()

    nfull = 12
    for b, length in enumerate(_SEQ_LENS):
        nworkers = length // _WROWS
        if nworkers == 0:
            continue

        @pl.when((bb == b) & (q < nworkers))
        def _(b=b):
            pltpu.async_copy(
                xs[b].at[pl.ds(jlo, 16), :],
                big.at[pl.ds(2 * _CR, 16), :],
                sem_in,
            ).wait()
            pltpu.async_copy(
                big.at[pl.ds(2 * _CR, 16), :],
                out.at[b, pl.ds(jlo, 16), :],
                sem_out,
            )

            def _step(k, carry):
                base = lax.rem(k, 3) * _CR
                j = jlo + 16 + k * _CR

                @pl.when(k == 2)
                def _():
                    _drain(sem_out, 16)

                @pl.when(k >= 3)
                def _():
                    _drain(sem_out, _CR)

                pltpu.async_copy(
                    xs[b].at[pl.ds(j, _CR), :],
                    big.at[pl.ds(base, _CR), :],
                    sem_in,
                ).wait()
                pltpu.async_copy(
                    big.at[pl.ds(base, _CR), :],
                    out.at[b, pl.ds(j, _CR), :],
                    sem_out,
                )
                return carry

            lax.fori_loop(0, nfull, _step, 0)
            _drain(sem_out, _CR)
            j = jlo + 16 + nfull * _CR
            pltpu.async_copy(
                xs[b].at[pl.ds(j, 16), :], big.at[pl.ds(0, 16), :], sem_in
            ).wait()
            pltpu.async_copy(
                big.at[pl.ds(0, 16), :], out.at[b, pl.ds(j, 16), :], sem_out
            )
            _drain(sem_out, _WROWS - 16 - 10 * _CR)

    @pl.when(q * _WROWS + 256 * bb >= _MAX)
    def _():
        _fill_zero(0, 16)
        pltpu.async_copy(
            big.at[pl.ds(0, 16), :],
            out.at[bb, pl.ds(jlo, 16), :],
            sem_z,
        )
        _fill_zero(16, _ROWS - 16)
        for t in range(4):
            pltpu.async_copy(
                big.at[pl.ds(0, _ROWS), :],
                out.at[bb, pl.ds(jlo + 16 + t * _ROWS, _ROWS), :],
                sem_z,
            )
        pltpu.async_copy(
            big.at[pl.ds(0, 16), :],
            out.at[bb, pl.ds(jlo + _WROWS - 16, 16), :],
            sem_z,
        )
        _drain(sem_z, 16)
        for t in range(4):
            _drain(sem_z, _ROWS)
        _drain(sem_z, 16)

    for b, length in enumerate(_SEQ_LENS):
        if length % _WROWS == 0:
            continue
        w = 4 * b + length // _WROWS
        j0 = (length // _WROWS) * _WROWS

        @pl.when(wid == w)
        def _(b=b, j0=j0):
            chunks = [(j0 + t * _CR, _CR) for t in range(6)] + [
                (j0 + 6 * _CR, 16)
            ]
            opend = [None] * len(chunks)
            for k, (j, m) in enumerate(chunks):
                base = (k % 2) * _CR
                if k >= 2:
                    opend[k - 2].wait()
                pltpu.async_copy(
                    xs[b].at[pl.ds(j, m), :],
                    big.at[pl.ds(base, m), :],
                    sem_in,
                ).wait()
                opend[k] = pltpu.async_copy(
                    big.at[pl.ds(base, m), :],
                    out.at[b, pl.ds(j, m), :],
                    sem_out,
                )
            _fill_zero(2 * _CR, _ROWS - 2 * _CR)
            zpend = []
            z0 = j0 + 256
            zchunks = [(z0 + t * _CR, _CR) for t in range(6)] + [
                (z0 + 6 * _CR, 16)
            ]
            for j, m in zchunks:
                zpend.append(
                    pltpu.async_copy(
                        big.at[pl.ds(2 * _CR, m), :],
                        out.at[b, pl.ds(j, m), :],
                        sem_z,
                    )
                )
            for p in opend[-2:]:
                p.wait()
            for p in zpend:
                p.wait()


@functools.cache
def _get_padder():
    return pl.kernel(
        _pad_body,
        out_type=jax.ShapeDtypeStruct((_B, _MAX, _D), jnp.float32),
        mesh=plsc.VectorSubcoreMesh(
            core_axis_name="c",
            subcore_axis_name="s",
            num_cores=_NC,
            num_subcores=_NS,
        ),
        scratch_types=[
            pltpu.VMEM((_ROWS, _D), jnp.float32),
            pltpu.SemaphoreType.DMA,
            pltpu.SemaphoreType.DMA,
            pltpu.SemaphoreType.DMA,
        ],
    )


def kernel(x0, x1, x2, x3, x4, x5, x6, x7):
    return _get_padder()(x0, x1, x2, x3, x4, x5, x6, x7)


# speedup vs baseline: 1.0057x; 1.0057x over previous
import functools

import jax
import jax.numpy as jnp
from jax import lax
from jax.experimental import pallas as pl
from jax.experimental.pallas import tpu as pltpu
from jax.experimental.pallas import tpu_sc as plsc

_SEQ_LENS = (2048, 1792, 1536, 1280, 1024, 768, 512, 256)
_B = 8
_MAX = 2048
_D = 1024
_NC = 2
_NS = 16
_NW = _NC * _NS
_WROWS = _B * _MAX // _NW
_ROWS = 120
_CR = 40


def _pad_body(x0, x1, x2, x3, x4, x5, x6, x7, out, big, sem_z, sem_in,
              sem_out):
    xs = (x0, x1, x2, x3, x4, x5, x6, x7)
    wid = lax.axis_index("s") * _NC + lax.axis_index("c")
    bb = wid // 4
    q = wid % 4
    jlo = q * _WROWS

    def _fill_zero(row0, nrows):
        def _row(r, carry):
            for cc in range(_D // 16):
                big[r, pl.ds(cc * 16, 16)] = jnp.zeros((16,), jnp.float32)
            return carry

        lax.fori_loop(row0, row0 + nrows, _row, 0)

    def _drain(sem, nrows):
        pltpu.make_async_copy(
            x0.at[pl.ds(0, nrows), :], big.at[pl.ds(0, nrows), :], sem
        ).wait()

    nfull = 12
    for b, length in enumerate(_SEQ_LENS):
        nworkers = length // _WROWS
        if nworkers == 0:
            continue

        @pl.when((bb == b) & (q < nworkers))
        def _(b=b):
            def _step(k, carry):
                base = lax.rem(k, 3) * _CR
                j = jlo + k * _CR

                @pl.when(k >= 3)
                def _():
                    _drain(sem_out, _CR)

                pltpu.async_copy(
                    xs[b].at[pl.ds(j, _CR), :],
                    big.at[pl.ds(base, _CR), :],
                    sem_in,
                ).wait()
                pltpu.async_copy(
                    big.at[pl.ds(base, _CR), :],
                    out.at[b, pl.ds(j, _CR), :],
                    sem_out,
                )
                return carry

            lax.fori_loop(0, nfull, _step, 0)
            _drain(sem_out, _CR)
            j = jlo + nfull * _CR
            pltpu.async_copy(
                xs[b].at[pl.ds(j, 32), :], big.at[pl.ds(0, 32), :], sem_in
            ).wait()
            pltpu.async_copy(
                big.at[pl.ds(0, 32), :], out.at[b, pl.ds(j, 32), :], sem_out
            )
            _drain(sem_out, _WROWS - 10 * _CR)

    @pl.when(q * _WROWS + 256 * bb >= _MAX)
    def _():
        _fill_zero(0, _CR)
        pltpu.async_copy(
            big.at[pl.ds(0, _CR), :],
            out.at[bb, pl.ds(jlo, _CR), :],
            sem_z,
        )
        _fill_zero(_CR, _ROWS - _CR)
        for t in range(3):
            pltpu.async_copy(
                big.at[pl.ds(0, _ROWS), :],
                out.at[bb, pl.ds(jlo + _CR + t * _ROWS, _ROWS), :],
                sem_z,
            )
        rem = _WROWS - _CR - 3 * _ROWS
        pltpu.async_copy(
            big.at[pl.ds(0, rem), :],
            out.at[bb, pl.ds(jlo + _WROWS - rem, rem), :],
            sem_z,
        )
        _drain(sem_z, _CR)
        for t in range(3):
            _drain(sem_z, _ROWS)
        _drain(sem_z, rem)

    for b, length in enumerate(_SEQ_LENS):
        if length % _WROWS == 0:
            continue
        w = 4 * b + length // _WROWS
        j0 = (length // _WROWS) * _WROWS

        @pl.when(wid == w)
        def _(b=b, j0=j0):
            chunks = [(j0 + t * _CR, _CR) for t in range(6)] + [
                (j0 + 6 * _CR, 16)
            ]
            opend = [None] * len(chunks)
            for k, (j, m) in enumerate(chunks):
                base = (k % 2) * _CR
                if k >= 2:
                    opend[k - 2].wait()
                pltpu.async_copy(
                    xs[b].at[pl.ds(j, m), :],
                    big.at[pl.ds(base, m), :],
                    sem_in,
                ).wait()
                opend[k] = pltpu.async_copy(
                    big.at[pl.ds(base, m), :],
                    out.at[b, pl.ds(j, m), :],
                    sem_out,
                )
            _fill_zero(2 * _CR, _ROWS - 2 * _CR)
            zpend = []
            z0 = j0 + 256
            zchunks = [(z0 + t * _CR, _CR) for t in range(6)] + [
                (z0 + 6 * _CR, 16)
            ]
            for j, m in zchunks:
                zpend.append(
                    pltpu.async_copy(
                        big.at[pl.ds(2 * _CR, m), :],
                        out.at[b, pl.ds(j, m), :],
                        sem_z,
                    )
                )
            for p in opend[-2:]:
                p.wait()
            for p in zpend:
                p.wait()


@functools.cache
def _get_padder():
    return pl.kernel(
        _pad_body,
        out_type=jax.ShapeDtypeStruct((_B, _MAX, _D), jnp.float32),
        mesh=plsc.VectorSubcoreMesh(
            core_axis_name="c",
            subcore_axis_name="s",
            num_cores=_NC,
            num_subcores=_NS,
        ),
        scratch_types=[
            pltpu.VMEM((_ROWS, _D), jnp.float32),
            pltpu.SemaphoreType.DMA,
            pltpu.SemaphoreType.DMA,
            pltpu.SemaphoreType.DMA,
        ],
    )


def kernel(x0, x1, x2, x3, x4, x5, x6, x7):
    return _get_padder()(x0, x1, x2, x3, x4, x5, x6, x7)
